# baseline (speedup 1.0000x reference)
import functools

import jax
import jax.numpy as jnp
from jax import lax
from jax.experimental import pallas as pl
from jax.experimental.pallas import tpu as pltpu
from jax.experimental.pallas import tpu_sc as plsc

EPS = 1e-6
BN_EPS = 1e-5
KNN = 20
B = 8
NF = 32
ND = 3
N = 1024
CO = 32
DM = ND * CO
TW = 2 * DM
GW = 256
ROWS = B * N * KNN
TILE = 64
RT = TILE * KNN
NTILES = (B * N) // TILE


def _setup_mats(W_feat, W_dir):
    eye3 = jnp.eye(ND, dtype=jnp.float32)

    def blk(w):
        m = jnp.einsum('oc,de->cdeo', w, eye3)
        return m.reshape(NF * ND, ND * CO)

    w1f, w2f = W_feat[:, :NF], W_feat[:, NF:]
    w1d, w2d = W_dir[:, :NF], W_dir[:, NF:]
    z = jnp.zeros((NF * ND, 128 - DM), jnp.float32)
    wn = jnp.concatenate([blk(w1f), z, blk(w1d), z], axis=1)
    wc = jnp.concatenate([blk(w2f - w1f), blk(w2d - w1d)], axis=1)

    r = jnp.arange(RT)
    sel_e = (r[:, None] // KNN == jnp.arange(TILE)[None, :]
             ).astype(jnp.float32)
    sel_s = sel_e.T
    d3 = (jnp.arange(DM)[:, None] % CO == jnp.arange(CO)[None, :]
          ).astype(jnp.float32)
    d3t = d3.T
    return wn, wc, sel_e, sel_s, d3, d3t


def _mm(a, b):
    return lax.dot_general(a, b, (((1,), (0,)), ((), ())),
                           preferred_element_type=jnp.float32)


def _stage_a_body(x_ref, wn_ref, wc_ref, idx_ref, tabn_ref, tabc_ref):
    xr = x_ref[0]
    xf = xr.reshape(NF * ND, N)
    xsq = xf * xf
    xx_row = jnp.sum(xsq, axis=0, keepdims=True)
    gram = lax.dot_general(xf, xf, (((0,), (0,)), ((), ())),
                           preferred_element_type=jnp.float32)
    ones = jnp.ones((NF * ND, 1), jnp.float32)
    xx_col = lax.dot_general(xsq, ones, (((0,), (0,)), ((), ())),
                             preferred_element_type=jnp.float32)
    pw = 2.0 * gram - xx_row - xx_col

    lane_iota = lax.broadcasted_iota(jnp.int32, (N, N), 1)
    neg_inf = jnp.float32(-jnp.inf)
    boff = pl.program_id(0) * N
    cols = []
    for _ in range(KNN):
        it = jnp.argmax(pw, axis=1).reshape(N, 1).astype(jnp.int32)
        cols.append(it + boff)
        pw = jnp.where(lane_iota == it, neg_inf, pw)
    idx_ref[0] = jnp.concatenate(cols, axis=1)

    xt_w = lax.dot_general(xf, wn_ref[...], (((0,), (0,)), ((), ())),
                           preferred_element_type=jnp.float32)
    tabn_ref[0] = xt_w
    tabc_ref[0] = lax.dot_general(xf, wc_ref[...], (((0,), (0,)), ((), ())),
                                  preferred_element_type=jnp.float32)


def _stage_a(x, wn, wc):
    nb = x.shape[0]
    return pl.pallas_call(
        _stage_a_body,
        grid=(nb,),
        in_specs=[
            pl.BlockSpec((1, NF, ND, N), lambda b: (b, 0, 0, 0)),
            pl.BlockSpec((NF * ND, GW), lambda b: (0, 0)),
            pl.BlockSpec((NF * ND, TW), lambda b: (0, 0)),
        ],
        out_specs=[
            pl.BlockSpec((1, N, KNN), lambda b: (b, 0, 0)),
            pl.BlockSpec((1, N, GW), lambda b: (b, 0, 0)),
            pl.BlockSpec((1, N, TW), lambda b: (b, 0, 0)),
        ],
        out_shape=[
            jax.ShapeDtypeStruct((nb, N, KNN), jnp.int32),
            jax.ShapeDtypeStruct((nb, N, GW), jnp.float32),
            jax.ShapeDtypeStruct((nb, N, TW), jnp.float32),
        ],
    )(x, wn, wc)


def _sc_gather(tab, fidx):
    info = plsc.get_sparse_core_info()
    nw = info.num_cores * info.num_subcores
    nrows = fidx.shape[0]
    rows_per_w = nrows // nw
    chunk = 128
    nchunks = rows_per_w // chunk
    mesh = plsc.VectorSubcoreMesh(core_axis_name="c", subcore_axis_name="s")

    nbuf = 3

    @functools.partial(
        pl.kernel,
        mesh=mesh,
        out_type=jax.ShapeDtypeStruct((nrows, GW), jnp.float32),
        scratch_types=[
            pltpu.VMEM((rows_per_w,), jnp.int32),
            pltpu.VMEM((chunk, GW), jnp.float32),
            pltpu.VMEM((chunk, GW), jnp.float32),
            pltpu.VMEM((chunk, GW), jnp.float32),
            pltpu.SemaphoreType.DMA,
            pltpu.SemaphoreType.DMA,
            pltpu.SemaphoreType.DMA,
            pltpu.SemaphoreType.DMA,
            pltpu.SemaphoreType.DMA,
            pltpu.SemaphoreType.DMA,
        ],
    )
    def gather_k(tab_hbm, idx_hbm, out_hbm, idx_v, r0, r1, r2,
                 g0, g1, g2, s0, s1, s2):
        rows = [r0, r1, r2]
        gsem = [g0, g1, g2]
        ssem = [s0, s1, s2]
        wid = lax.axis_index("s") * info.num_cores + lax.axis_index("c")
        base = wid * rows_per_w
        pltpu.sync_copy(idx_hbm.at[pl.ds(base, rows_per_w)], idx_v)

        def start_gather(c, b):
            return pltpu.async_copy(
                tab_hbm.at[idx_v.at[pl.ds(c * chunk, chunk)]],
                rows[b], gsem[b])

        def start_store(c, b):
            return pltpu.async_copy(
                rows[b], out_hbm.at[pl.ds(base + c * chunk, chunk)], ssem[b])

        gctx = [None] * nbuf
        sctx = [None] * nbuf
        for c in range(min(nbuf, nchunks)):
            gctx[c] = start_gather(c, c)
        for c in range(nchunks):
            b = c % nbuf
            gctx[b].wait()
            sctx[b] = start_store(c, b)
            nxt = c + nbuf
            if nxt < nchunks:
                sctx[b].wait()
                gctx[b] = start_gather(nxt, b)
        for c in range(max(0, nchunks - nbuf), nchunks):
            sctx[c % nbuf].wait()

    return gather_k(tab, fidx)


def _pass1_body(g_ref, c_ref, se_ref, d3_ref, o_ref):
    g = g_ref[:, 0:DM]
    cen_e = _mm(se_ref[...], c_ref[:, 0:DM])
    p = g + cen_e
    nsq = _mm(p * p, d3_ref[...])
    nrm = jnp.sqrt(nsq) + EPS
    sn = jnp.sum(nrm, axis=0, keepdims=True)
    sn2 = jnp.sum(nrm * nrm, axis=0, keepdims=True)
    part = jnp.concatenate([sn, sn2, jnp.zeros((1, 64), jnp.float32)], axis=1)

    @pl.when(pl.program_id(0) == 0)
    def _init():
        o_ref[...] = part

    @pl.when(pl.program_id(0) != 0)
    def _acc():
        o_ref[...] += part


def _pass1(gath2, tabc2, sel_e, d3):
    return pl.pallas_call(
        _pass1_body,
        grid=(gath2.shape[0] // RT,),
        in_specs=[
            pl.BlockSpec((RT, 128), lambda i: (i, 0)),
            pl.BlockSpec((TILE, TW), lambda i: (i, 0)),
            pl.BlockSpec((RT, TILE), lambda i: (0, 0)),
            pl.BlockSpec((DM, CO), lambda i: (0, 0)),
        ],
        out_specs=pl.BlockSpec((1, 128), lambda i: (0, 0)),
        out_shape=jax.ShapeDtypeStruct((1, 128), jnp.float32),
    )(gath2, tabc2, sel_e, d3)


def _pass2_body(g_ref, c_ref, s_ref, gam_ref, bet_ref, se_ref, ss_ref,
                d3_ref, d3t_ref, o_ref):
    cen_e = _mm(se_ref[...], c_ref[...])
    p = g_ref[:, 0:DM] + cen_e[:, 0:DM]
    d = g_ref[:, 128:128 + DM] + cen_e[:, DM:TW]

    d3 = d3_ref[...]
    nsq = _mm(p * p, d3)
    nrm = jnp.sqrt(nsq) + EPS
    s = _mm(p * d, d3)
    dn = _mm(d * d, d3)

    stats = s_ref[...]
    cnt = jnp.float32(ROWS)
    mu = stats[0:1, 0:CO] / cnt
    ex2 = stats[0:1, CO:2 * CO] / cnt
    var = ex2 - mu * mu
    gam = gam_ref[...]
    bet = bet_ref[...]

    nbn = gam * (nrm - mu) / jnp.sqrt(var + BN_EPS) + bet
    cfac = nbn / nrm
    dot = cfac * s
    coef = jnp.where(dot >= 0, jnp.float32(0.0), 0.8 * dot / (dn + EPS))
    d3t = d3t_ref[...]
    y = _mm(cfac, d3t) * p - _mm(coef, d3t) * d
    o_ref[...] = _mm(ss_ref[...], y) * jnp.float32(1.0 / KNN)


def _pass2(gath2, tabc2, stats, gam, bet, sel_e, sel_s, d3, d3t):
    return pl.pallas_call(
        _pass2_body,
        grid=(gath2.shape[0] // RT,),
        in_specs=[
            pl.BlockSpec((RT, GW), lambda i: (i, 0)),
            pl.BlockSpec((TILE, TW), lambda i: (i, 0)),
            pl.BlockSpec((1, 128), lambda i: (0, 0)),
            pl.BlockSpec((1, CO), lambda i: (0, 0)),
            pl.BlockSpec((1, CO), lambda i: (0, 0)),
            pl.BlockSpec((RT, TILE), lambda i: (0, 0)),
            pl.BlockSpec((TILE, RT), lambda i: (0, 0)),
            pl.BlockSpec((DM, CO), lambda i: (0, 0)),
            pl.BlockSpec((CO, DM), lambda i: (0, 0)),
        ],
        out_specs=pl.BlockSpec((TILE, DM), lambda i: (i, 0)),
        out_shape=jax.ShapeDtypeStruct((gath2.shape[0] // KNN, DM),
                                       jnp.float32),
    )(gath2, tabc2, stats, gam, bet, sel_e, sel_s, d3, d3t)


def kernel(x, W_feat, W_dir, gamma, beta):
    wn, wc, sel_e, sel_s, d3, d3t = _setup_mats(W_feat, W_dir)
    nsplit = 4
    hb = B // nsplit
    gam, bet = gamma.reshape(1, CO), beta.reshape(1, CO)

    tabs, gaths = [], []
    for h in range(nsplit):
        idx20, tabn, tabc = _stage_a(x[h * hb:(h + 1) * hb], wn, wc)
        tabs.append(tabc.reshape(hb * N, TW))
        gaths.append(_sc_gather(tabn.reshape(hb * N, GW),
                                idx20.reshape(hb * N * KNN)))
    stats = sum(_pass1(gaths[h], tabs[h], sel_e, d3)
                for h in range(nsplit))
    outs = [_pass2(gaths[h], tabs[h], stats, gam, bet,
                   sel_e, sel_s, d3, d3t).reshape(hb, N, ND, CO)
            for h in range(nsplit)]
    return jnp.concatenate(outs, axis=0).transpose(0, 3, 2, 1)


# speedup vs baseline: 1.0874x; 1.0874x over previous
import functools

import jax
import jax.numpy as jnp
from jax import lax
from jax.experimental import pallas as pl
from jax.experimental.pallas import tpu as pltpu
from jax.experimental.pallas import tpu_sc as plsc

EPS = 1e-6
BN_EPS = 1e-5
KNN = 20
B = 8
NF = 32
ND = 3
N = 1024
CO = 32
DM = ND * CO
TW = 2 * DM
GW = 256
ROWS = B * N * KNN
TILE = 64
RT = TILE * KNN
NTILES = (B * N) // TILE


def _setup_mats(W_feat, W_dir):
    eye3 = jnp.eye(ND, dtype=jnp.float32)

    def blk(w):
        m = jnp.einsum('oc,de->cdeo', w, eye3)
        return m.reshape(NF * ND, ND * CO)

    w1f, w2f = W_feat[:, :NF], W_feat[:, NF:]
    w1d, w2d = W_dir[:, :NF], W_dir[:, NF:]
    z = jnp.zeros((NF * ND, 128 - DM), jnp.float32)
    wn = jnp.concatenate([blk(w1f), z, blk(w1d), z], axis=1)
    wc = jnp.concatenate([blk(w2f - w1f), blk(w2d - w1d)], axis=1)

    r = jnp.arange(RT)
    sel_e = (r[:, None] // KNN == jnp.arange(TILE)[None, :]
             ).astype(jnp.float32)
    sel_s = sel_e.T
    d3 = (jnp.arange(DM)[:, None] % CO == jnp.arange(CO)[None, :]
          ).astype(jnp.float32)
    d3t = d3.T
    return wn, wc, sel_e, sel_s, d3, d3t


def _mm(a, b):
    return lax.dot_general(a, b, (((1,), (0,)), ((), ())),
                           preferred_element_type=jnp.float32)


def _stage_a_body(x_ref, wn_ref, wc_ref, idx_ref, tabn_ref, tabc_ref):
    xr = x_ref[0]
    xf = xr.reshape(NF * ND, N)
    xsq = xf * xf
    xx_row = jnp.sum(xsq, axis=0, keepdims=True)
    gram = lax.dot_general(xf, xf, (((0,), (0,)), ((), ())),
                           preferred_element_type=jnp.float32)
    ones = jnp.ones((NF * ND, 1), jnp.float32)
    xx_col = lax.dot_general(xsq, ones, (((0,), (0,)), ((), ())),
                             preferred_element_type=jnp.float32)
    pw = 2.0 * gram - xx_row - xx_col

    lane_iota = lax.broadcasted_iota(jnp.int32, (N, N), 1)
    neg_inf = jnp.float32(-jnp.inf)
    boff = pl.program_id(0) * N
    cols = []
    for _ in range(KNN):
        it = jnp.argmax(pw, axis=1).reshape(N, 1).astype(jnp.int32)
        cols.append(it + boff)
        pw = jnp.where(lane_iota == it, neg_inf, pw)
    idx_ref[0] = jnp.concatenate(cols, axis=1)

    xt_w = lax.dot_general(xf, wn_ref[...], (((0,), (0,)), ((), ())),
                           preferred_element_type=jnp.float32)
    tabn_ref[0] = xt_w
    tabc_ref[0] = lax.dot_general(xf, wc_ref[...], (((0,), (0,)), ((), ())),
                                  preferred_element_type=jnp.float32)


def _stage_a(x, wn, wc):
    nb = x.shape[0]
    return pl.pallas_call(
        _stage_a_body,
        grid=(nb,),
        in_specs=[
            pl.BlockSpec((1, NF, ND, N), lambda b: (b, 0, 0, 0)),
            pl.BlockSpec((NF * ND, GW), lambda b: (0, 0)),
            pl.BlockSpec((NF * ND, TW), lambda b: (0, 0)),
        ],
        out_specs=[
            pl.BlockSpec((1, N, KNN), lambda b: (b, 0, 0)),
            pl.BlockSpec((1, N, GW), lambda b: (b, 0, 0)),
            pl.BlockSpec((1, N, TW), lambda b: (b, 0, 0)),
        ],
        out_shape=[
            jax.ShapeDtypeStruct((nb, N, KNN), jnp.int32),
            jax.ShapeDtypeStruct((nb, N, GW), jnp.float32),
            jax.ShapeDtypeStruct((nb, N, TW), jnp.float32),
        ],
    )(x, wn, wc)


def _sc_gather(tab, fidx):
    info = plsc.get_sparse_core_info()
    nw = info.num_cores * info.num_subcores
    nrows = fidx.shape[0]
    rows_per_w = nrows // nw
    chunk = 128
    nchunks = rows_per_w // chunk
    mesh = plsc.VectorSubcoreMesh(core_axis_name="c", subcore_axis_name="s")

    nbuf = 3

    @functools.partial(
        pl.kernel,
        mesh=mesh,
        out_type=jax.ShapeDtypeStruct((nrows, GW), jnp.float32),
        scratch_types=[
            pltpu.VMEM((rows_per_w,), jnp.int32),
            pltpu.VMEM((chunk, GW), jnp.float32),
            pltpu.VMEM((chunk, GW), jnp.float32),
            pltpu.VMEM((chunk, GW), jnp.float32),
            pltpu.SemaphoreType.DMA,
            pltpu.SemaphoreType.DMA,
            pltpu.SemaphoreType.DMA,
            pltpu.SemaphoreType.DMA,
            pltpu.SemaphoreType.DMA,
            pltpu.SemaphoreType.DMA,
        ],
    )
    def gather_k(tab_hbm, idx_hbm, out_hbm, idx_v, r0, r1, r2,
                 g0, g1, g2, s0, s1, s2):
        rows = [r0, r1, r2]
        gsem = [g0, g1, g2]
        ssem = [s0, s1, s2]
        wid = lax.axis_index("s") * info.num_cores + lax.axis_index("c")
        base = wid * rows_per_w
        pltpu.sync_copy(idx_hbm.at[pl.ds(base, rows_per_w)], idx_v)

        def start_gather(c, b):
            return pltpu.async_copy(
                tab_hbm.at[idx_v.at[pl.ds(c * chunk, chunk)]],
                rows[b], gsem[b])

        def start_store(c, b):
            return pltpu.async_copy(
                rows[b], out_hbm.at[pl.ds(base + c * chunk, chunk)], ssem[b])

        gctx = [None] * nbuf
        sctx = [None] * nbuf
        for c in range(min(nbuf, nchunks)):
            gctx[c] = start_gather(c, c)
        for c in range(nchunks):
            b = c % nbuf
            gctx[b].wait()
            sctx[b] = start_store(c, b)
            nxt = c + nbuf
            if nxt < nchunks:
                sctx[b].wait()
                gctx[b] = start_gather(nxt, b)
        for c in range(max(0, nchunks - nbuf), nchunks):
            sctx[c % nbuf].wait()

    return gather_k(tab, fidx)


def _pass1_body(g_ref, c_ref, se_ref, d3_ref, o_ref):
    g = g_ref[:, 0:DM]
    cen_e = _mm(se_ref[...], c_ref[:, 0:DM])
    p = g + cen_e
    nsq = _mm(p * p, d3_ref[...])
    nrm = jnp.sqrt(nsq) + EPS
    sn = jnp.sum(nrm, axis=0, keepdims=True)
    sn2 = jnp.sum(nrm * nrm, axis=0, keepdims=True)
    part = jnp.concatenate([sn, sn2, jnp.zeros((1, 64), jnp.float32)], axis=1)

    @pl.when(pl.program_id(0) == 0)
    def _init():
        o_ref[...] = part

    @pl.when(pl.program_id(0) != 0)
    def _acc():
        o_ref[...] += part


def _pass1(gath2, tabc2, sel_e, d3):
    return pl.pallas_call(
        _pass1_body,
        grid=(gath2.shape[0] // RT,),
        in_specs=[
            pl.BlockSpec((RT, 128), lambda i: (i, 0)),
            pl.BlockSpec((TILE, TW), lambda i: (i, 0)),
            pl.BlockSpec((RT, TILE), lambda i: (0, 0)),
            pl.BlockSpec((DM, CO), lambda i: (0, 0)),
        ],
        out_specs=pl.BlockSpec((1, 128), lambda i: (0, 0)),
        out_shape=jax.ShapeDtypeStruct((1, 128), jnp.float32),
    )(gath2, tabc2, sel_e, d3)


def _pass2_body(g_ref, c_ref, s_ref, gam_ref, bet_ref, se_ref, ss_ref,
                d3_ref, d3t_ref, o_ref):
    cen_e = _mm(se_ref[...], c_ref[...])
    p = g_ref[:, 0:DM] + cen_e[:, 0:DM]
    d = g_ref[:, 128:128 + DM] + cen_e[:, DM:TW]

    d3 = d3_ref[...]
    nsq = _mm(p * p, d3)
    nrm = jnp.sqrt(nsq) + EPS
    s = _mm(p * d, d3)
    dn = _mm(d * d, d3)

    stats = s_ref[...]
    cnt = jnp.float32(ROWS)
    mu = stats[0:1, 0:CO] / cnt
    ex2 = stats[0:1, CO:2 * CO] / cnt
    var = ex2 - mu * mu
    gam = gam_ref[...]
    bet = bet_ref[...]

    nbn = gam * (nrm - mu) / jnp.sqrt(var + BN_EPS) + bet
    cfac = nbn / nrm
    dot = cfac * s
    coef = jnp.where(dot >= 0, jnp.float32(0.0), 0.8 * dot / (dn + EPS))
    d3t = d3t_ref[...]
    y = _mm(cfac, d3t) * p - _mm(coef, d3t) * d
    o_ref[...] = _mm(ss_ref[...], y) * jnp.float32(1.0 / KNN)


def _pass2(gath2, tabc2, stats, gam, bet, sel_e, sel_s, d3, d3t):
    return pl.pallas_call(
        _pass2_body,
        grid=(gath2.shape[0] // RT,),
        in_specs=[
            pl.BlockSpec((RT, GW), lambda i: (i, 0)),
            pl.BlockSpec((TILE, TW), lambda i: (i, 0)),
            pl.BlockSpec((1, 128), lambda i: (0, 0)),
            pl.BlockSpec((1, CO), lambda i: (0, 0)),
            pl.BlockSpec((1, CO), lambda i: (0, 0)),
            pl.BlockSpec((RT, TILE), lambda i: (0, 0)),
            pl.BlockSpec((TILE, RT), lambda i: (0, 0)),
            pl.BlockSpec((DM, CO), lambda i: (0, 0)),
            pl.BlockSpec((CO, DM), lambda i: (0, 0)),
        ],
        out_specs=pl.BlockSpec((TILE, DM), lambda i: (i, 0)),
        out_shape=jax.ShapeDtypeStruct((gath2.shape[0] // KNN, DM),
                                       jnp.float32),
    )(gath2, tabc2, stats, gam, bet, sel_e, sel_s, d3, d3t)


def kernel(x, W_feat, W_dir, gamma, beta):
    wn, wc, sel_e, sel_s, d3, d3t = _setup_mats(W_feat, W_dir)
    nsplit = 2
    hb = B // nsplit
    gam, bet = gamma.reshape(1, CO), beta.reshape(1, CO)

    tabs, gaths = [], []
    for h in range(nsplit):
        idx20, tabn, tabc = _stage_a(x[h * hb:(h + 1) * hb], wn, wc)
        tabs.append(tabc.reshape(hb * N, TW))
        gaths.append(_sc_gather(tabn.reshape(hb * N, GW),
                                idx20.reshape(hb * N * KNN)))
    stats = sum(_pass1(gaths[h], tabs[h], sel_e, d3)
                for h in range(nsplit))
    outs = [_pass2(gaths[h], tabs[h], stats, gam, bet,
                   sel_e, sel_s, d3, d3t).reshape(hb, N, ND, CO)
            for h in range(nsplit)]
    return jnp.concatenate(outs, axis=0).transpose(0, 3, 2, 1)
